# baseline (speedup 1.0000x reference)
import jax
import jax.numpy as jnp
from jax.experimental import pallas as pl
from jax.experimental.pallas import tpu as pltpu

N = 3276800
BLK = 2048
ROWS = 8
NB = N // (ROWS * BLK)


def _tc_body(cb_ref, ib_ref, g_ref, x_ref, o_ref):
    idx = g_ref[0] - 1
    c = jnp.take_along_axis(cb_ref[...], idx, axis=1)
    inv = jnp.take_along_axis(ib_ref[...], idx, axis=1)
    o_ref[0] = (x_ref[0] - c) * inv


@jax.jit
def _tc_run(x, group, cb8, ib8):
    x2 = x.reshape(NB, ROWS, BLK)
    g2 = group.reshape(NB, ROWS, BLK)
    out = pl.pallas_call(
        _tc_body,
        grid=(NB,),
        in_specs=[
            pl.BlockSpec((ROWS, 128), lambda i: (0, 0)),
            pl.BlockSpec((ROWS, 128), lambda i: (0, 0)),
            pl.BlockSpec((1, ROWS, BLK), lambda i: (i, 0, 0)),
            pl.BlockSpec((1, ROWS, BLK), lambda i: (i, 0, 0)),
        ],
        out_specs=pl.BlockSpec((1, ROWS, BLK), lambda i: (i, 0, 0)),
        out_shape=jax.ShapeDtypeStruct((NB, ROWS, BLK), jnp.float32),
        compiler_params=pltpu.CompilerParams(
            dimension_semantics=("arbitrary",)),
    )(cb8, ib8, g2, x2)
    return out.reshape(N)


def kernel(x, group, centers, scales):
    g = centers.shape[0]
    c_pad = jnp.zeros((128,), jnp.float32).at[:g].set(centers)
    i_pad = jnp.ones((128,), jnp.float32).at[:g].set(1.0 / scales)
    cb8 = jnp.broadcast_to(c_pad[None, :], (ROWS, 128))
    ib8 = jnp.broadcast_to(i_pad[None, :], (ROWS, 128))
    return _tc_run(x, group, cb8, ib8)
